# Initial kernel scaffold; baseline (speedup 1.0000x reference)
import jax
import jax.numpy as jnp
from jax.experimental import pallas as pl


def kernel(x, edge_index, Wp1, bp1, Wl1, bl1, Wr1, g1, be1, Wp2, bp2, Wl2, bl2, Wr2):
    raise NotImplementedError("write your pallas kernel here")


# speedup vs baseline: 4.5196x; 4.5196x over previous
import functools

import jax
import jax.numpy as jnp
from jax import lax
from jax.experimental import pallas as pl
from jax.experimental.pallas import tpu as pltpu
from jax.experimental.pallas import tpu_sc as plsc

N_NODES = 10000
N_EDGES = 320000
D = 128

NC, NS = 2, 16
APAD = 10240
ZB = APAD // NS
CH = 80
HOLE = APAD // 2
GLO, GHI = HOLE - 8, HOLE + 8
EPC = N_EDGES // NC
EPT = EPC // NS
NCH = EPT // CH

_mesh = plsc.VectorSubcoreMesh(core_axis_name="c", subcore_axis_name="s")


@functools.partial(
    pl.kernel,
    out_type=jax.ShapeDtypeStruct((NC, APAD, D), jnp.float32),
    mesh=_mesh,
    scratch_types=[
        pltpu.VMEM((CH,), jnp.int32),
        pltpu.VMEM((CH,), jnp.int32),
        pltpu.VMEM((CH, D), jnp.float32),
        pltpu.VMEM_SHARED((APAD, D), jnp.float32),
        pltpu.SemaphoreType.DMA,
    ],
)
def _sc_aggregate(xp_hbm, src_hbm, dst_hbm, zrow_hbm,
                  acc_out, src_v, dst_v, rows_v, acc_sh, sem):
    c = lax.axis_index("c")
    s = lax.axis_index("s")

    pltpu.sync_copy(zrow_hbm, rows_v)
    for k in range(ZB // CH):
        pltpu.sync_copy(rows_v, acc_sh.at[pl.ds(s * ZB + k * CH, CH)])
    plsc.subcore_barrier()

    ebase = c * EPC + s * EPT

    @pl.loop(0, NCH)
    def _(j):
        base = ebase + j * CH
        pltpu.sync_copy(src_hbm.at[pl.ds(base, CH)], src_v)
        pltpu.sync_copy(dst_hbm.at[pl.ds(base, CH)], dst_v)
        pltpu.async_copy(xp_hbm.at[src_v], rows_v, sem).wait()
        pltpu.sync_copy(rows_v, acc_sh.at[dst_v], add=True)

    plsc.subcore_barrier()
    for k in range(ZB // CH):
        pltpu.sync_copy(acc_sh.at[pl.ds(s * ZB + k * CH, CH)], rows_v)
        pltpu.sync_copy(rows_v, acc_out.at[c, pl.ds(s * ZB + k * CH, CH)])


@functools.partial(
    pl.kernel,
    out_type=jax.ShapeDtypeStruct((NC, APAD, D), jnp.float32),
    mesh=_mesh,
    scratch_types=[
        pltpu.VMEM((CH,), jnp.int32),
        pltpu.VMEM((CH, D), jnp.float32),
        pltpu.VMEM_SHARED((APAD, D), jnp.float32),
    ],
)
def _sc_count(dst_hbm, zrow_hbm, ones_hbm,
              cnt_out, dst_v, rows_v, cnt_sh):
    c = lax.axis_index("c")
    s = lax.axis_index("s")

    pltpu.sync_copy(zrow_hbm, rows_v)
    for k in range(ZB // CH):
        pltpu.sync_copy(rows_v, cnt_sh.at[pl.ds(s * ZB + k * CH, CH)])
    pltpu.sync_copy(ones_hbm, rows_v)
    plsc.subcore_barrier()

    ebase = c * EPC + s * EPT

    @pl.loop(0, NCH)
    def _(j):
        base = ebase + j * CH
        pltpu.sync_copy(dst_hbm.at[pl.ds(base, CH)], dst_v)
        pltpu.sync_copy(rows_v, cnt_sh.at[dst_v], add=True)

    plsc.subcore_barrier()
    for k in range(ZB // CH):
        pltpu.sync_copy(cnt_sh.at[pl.ds(s * ZB + k * CH, CH)], rows_v)
        pltpu.sync_copy(rows_v, cnt_out.at[c, pl.ds(s * ZB + k * CH, CH)])


_BR = 2000


def _dot(a, b):
    return jnp.dot(a, b, preferred_element_type=jnp.float32,
                   precision=lax.Precision.HIGHEST)


def _dense_relu(x, W, b):
    def body(x_ref, w_ref, b_ref, o_ref):
        o_ref[...] = jnp.maximum(_dot(x_ref[...], w_ref[...]) + b_ref[...], 0.0)

    return pl.pallas_call(
        body,
        grid=(N_NODES // _BR,),
        in_specs=[
            pl.BlockSpec((_BR, D), lambda i: (i, 0)),
            pl.BlockSpec((D, D), lambda i: (0, 0)),
            pl.BlockSpec((1, D), lambda i: (0, 0)),
        ],
        out_specs=pl.BlockSpec((_BR, D), lambda i: (i, 0)),
        out_shape=jax.ShapeDtypeStruct((N_NODES, D), jnp.float32),
    )(x, W, b.reshape(1, D))


def _combine_mid(part, cnt1, xp, Wl, bl, Wr, g, be, Wp2, bp2):
    def body(p_ref, c_ref, xp_ref, wl_ref, bl_ref, wr_ref, g_ref, be_ref,
             wp_ref, bp_ref, o_ref):
        summed = p_ref[0] + p_ref[1]
        cnt = c_ref[0] + c_ref[1]
        mean = summed / jnp.maximum(cnt, 1.0)
        t = _dot(mean, wl_ref[...]) + bl_ref[...] + _dot(xp_ref[...], wr_ref[...])
        t = jnp.maximum(t, 0.0)
        mu = jnp.mean(t, axis=-1, keepdims=True)
        var = jnp.mean((t - mu) * (t - mu), axis=-1, keepdims=True)
        h = (t - mu) * lax.rsqrt(var + 1e-5) * g_ref[...] + be_ref[...]
        o_ref[...] = jnp.maximum(_dot(h, wp_ref[...]) + bp_ref[...], 0.0)

    return pl.pallas_call(
        body,
        grid=(N_NODES // _BR,),
        in_specs=[
            pl.BlockSpec((NC, _BR, D), lambda i: (0, i, 0)),
            pl.BlockSpec((NC, _BR, 1), lambda i: (0, i, 0)),
            pl.BlockSpec((_BR, D), lambda i: (i, 0)),
            pl.BlockSpec((D, D), lambda i: (0, 0)),
            pl.BlockSpec((1, D), lambda i: (0, 0)),
            pl.BlockSpec((D, D), lambda i: (0, 0)),
            pl.BlockSpec((1, D), lambda i: (0, 0)),
            pl.BlockSpec((1, D), lambda i: (0, 0)),
            pl.BlockSpec((D, D), lambda i: (0, 0)),
            pl.BlockSpec((1, D), lambda i: (0, 0)),
        ],
        out_specs=pl.BlockSpec((_BR, D), lambda i: (i, 0)),
        out_shape=jax.ShapeDtypeStruct((N_NODES, D), jnp.float32),
    )(part, cnt1, xp, Wl, bl.reshape(1, D), Wr, g.reshape(1, D),
      be.reshape(1, D), Wp2, bp2.reshape(1, D))


def _combine_final(part, cnt1, xp, Wl, bl, Wr):
    def body(p_ref, c_ref, xp_ref, wl_ref, bl_ref, wr_ref, o_ref):
        summed = p_ref[0] + p_ref[1]
        cnt = c_ref[0] + c_ref[1]
        mean = summed / jnp.maximum(cnt, 1.0)
        o_ref[...] = (_dot(mean, wl_ref[...]) + bl_ref[...]
                      + _dot(xp_ref[...], wr_ref[...]))

    return pl.pallas_call(
        body,
        grid=(N_NODES // _BR,),
        in_specs=[
            pl.BlockSpec((NC, _BR, D), lambda i: (0, i, 0)),
            pl.BlockSpec((NC, _BR, 1), lambda i: (0, i, 0)),
            pl.BlockSpec((_BR, D), lambda i: (i, 0)),
            pl.BlockSpec((D, D), lambda i: (0, 0)),
            pl.BlockSpec((1, D), lambda i: (0, 0)),
            pl.BlockSpec((D, D), lambda i: (0, 0)),
        ],
        out_specs=pl.BlockSpec((_BR, D), lambda i: (i, 0)),
        out_shape=jax.ShapeDtypeStruct((N_NODES, D), jnp.float32),
    )(part, cnt1, xp, Wl, bl.reshape(1, D), Wr)


def _unmap(a):
    return jnp.concatenate(
        [a[:, :GLO], a[:, GHI:GHI + (N_NODES - GLO)]], axis=1)


def kernel(x, edge_index, Wp1, bp1, Wl1, bl1, Wr1, g1, be1,
           Wp2, bp2, Wl2, bl2, Wr2):
    src = edge_index[0].astype(jnp.int32)
    dst = edge_index[1].astype(jnp.int32)
    dst2 = dst + jnp.where(dst >= GLO, 16, 0).astype(jnp.int32)
    zrow = jnp.zeros((CH, D), jnp.float32)
    ones = jnp.ones((CH, D), jnp.float32)

    cnt = _sc_count(dst2, zrow, ones)
    cnt1 = _unmap(cnt[:, :, :1])

    xp1 = _dense_relu(x, Wp1, bp1)
    part1 = _unmap(_sc_aggregate(xp1, src, dst2, zrow))
    xp2 = _combine_mid(part1, cnt1, xp1, Wl1, bl1, Wr1, g1, be1, Wp2, bp2)
    part2 = _unmap(_sc_aggregate(xp2, src, dst2, zrow))
    out = _combine_final(part2, cnt1, xp2, Wl2, bl2, Wr2)
    return out


# speedup vs baseline: 6.4131x; 1.4190x over previous
import functools

import jax
import jax.numpy as jnp
from jax import lax
from jax.experimental import pallas as pl
from jax.experimental.pallas import tpu as pltpu
from jax.experimental.pallas import tpu_sc as plsc

N_NODES = 10000
N_EDGES = 320000
D = 128

NC, NS = 2, 16
APAD = 10240
ZB = APAD // NS
CH = 80
HOLE = APAD // 2
GLO, GHI = HOLE - 8, HOLE + 8
EPC = N_EDGES // NC
EPT = EPC // NS
NCH = EPT // CH

_mesh = plsc.VectorSubcoreMesh(core_axis_name="c", subcore_axis_name="s")


@functools.partial(
    pl.kernel,
    out_type=jax.ShapeDtypeStruct((NC, APAD, D), jnp.float32),
    mesh=_mesh,
    scratch_types=[
        pltpu.VMEM((CH,), jnp.int32),
        pltpu.VMEM((CH,), jnp.int32),
        pltpu.VMEM((CH,), jnp.int32),
        pltpu.VMEM((CH,), jnp.int32),
        pltpu.VMEM((CH, D), jnp.float32),
        pltpu.VMEM((CH, D), jnp.float32),
        pltpu.VMEM_SHARED((APAD, D), jnp.float32),
        pltpu.SemaphoreType.DMA,
        pltpu.SemaphoreType.DMA,
    ],
)
def _sc_aggregate(xp_hbm, src_hbm, dst_hbm, zrow_hbm,
                  acc_out, src_a, dst_a, src_b, dst_b,
                  rows_a, rows_b, acc_sh, sem_a, sem_b):
    c = lax.axis_index("c")
    s = lax.axis_index("s")

    pltpu.sync_copy(zrow_hbm, rows_a)
    for k in range(ZB // CH):
        pltpu.sync_copy(rows_a, acc_sh.at[pl.ds(s * ZB + k * CH, CH)])
    plsc.subcore_barrier()

    ebase = c * EPC + s * EPT

    pltpu.sync_copy(src_hbm.at[pl.ds(ebase, CH)], src_a)
    pltpu.sync_copy(dst_hbm.at[pl.ds(ebase, CH)], dst_a)
    ga = pltpu.async_copy(xp_hbm.at[src_a], rows_a, sem_a)

    @pl.loop(0, (NCH - 1) // 2)
    def _(j2):
        base = ebase + (2 * j2) * CH
        pltpu.sync_copy(src_hbm.at[pl.ds(base + CH, CH)], src_b)
        pltpu.sync_copy(dst_hbm.at[pl.ds(base + CH, CH)], dst_b)
        gb = pltpu.async_copy(xp_hbm.at[src_b], rows_b, sem_b)
        ga.wait()
        pltpu.sync_copy(rows_a, acc_sh.at[dst_a], add=True)
        pltpu.sync_copy(src_hbm.at[pl.ds(base + 2 * CH, CH)], src_a)
        pltpu.sync_copy(dst_hbm.at[pl.ds(base + 2 * CH, CH)], dst_a)
        pltpu.async_copy(xp_hbm.at[src_a], rows_a, sem_a)
        gb.wait()
        pltpu.sync_copy(rows_b, acc_sh.at[dst_b], add=True)

    ga.wait()
    pltpu.sync_copy(rows_a, acc_sh.at[dst_a], add=True)

    plsc.subcore_barrier()
    for k in range(ZB // CH):
        pltpu.sync_copy(acc_sh.at[pl.ds(s * ZB + k * CH, CH)], rows_a)
        pltpu.sync_copy(rows_a, acc_out.at[c, pl.ds(s * ZB + k * CH, CH)])


@functools.partial(
    pl.kernel,
    out_type=jax.ShapeDtypeStruct((NC, APAD, D), jnp.float32),
    mesh=_mesh,
    scratch_types=[
        pltpu.VMEM((CH,), jnp.int32),
        pltpu.VMEM((CH, D), jnp.float32),
        pltpu.VMEM_SHARED((APAD, D), jnp.float32),
    ],
)
def _sc_count(dst_hbm, zrow_hbm, ones_hbm,
              cnt_out, dst_v, rows_v, cnt_sh):
    c = lax.axis_index("c")
    s = lax.axis_index("s")

    pltpu.sync_copy(zrow_hbm, rows_v)
    for k in range(ZB // CH):
        pltpu.sync_copy(rows_v, cnt_sh.at[pl.ds(s * ZB + k * CH, CH)])
    pltpu.sync_copy(ones_hbm, rows_v)
    plsc.subcore_barrier()

    ebase = c * EPC + s * EPT

    @pl.loop(0, NCH)
    def _(j):
        base = ebase + j * CH
        pltpu.sync_copy(dst_hbm.at[pl.ds(base, CH)], dst_v)
        pltpu.sync_copy(rows_v, cnt_sh.at[dst_v], add=True)

    plsc.subcore_barrier()
    for k in range(ZB // CH):
        pltpu.sync_copy(cnt_sh.at[pl.ds(s * ZB + k * CH, CH)], rows_v)
        pltpu.sync_copy(rows_v, cnt_out.at[c, pl.ds(s * ZB + k * CH, CH)])


_BR = 2000


def _dot(a, b):
    return jnp.dot(a, b, preferred_element_type=jnp.float32,
                   precision=lax.Precision.HIGHEST)


def _dense_relu(x, W, b):
    def body(x_ref, w_ref, b_ref, o_ref):
        o_ref[...] = jnp.maximum(_dot(x_ref[...], w_ref[...]) + b_ref[...], 0.0)

    return pl.pallas_call(
        body,
        grid=(N_NODES // _BR,),
        in_specs=[
            pl.BlockSpec((_BR, D), lambda i: (i, 0)),
            pl.BlockSpec((D, D), lambda i: (0, 0)),
            pl.BlockSpec((1, D), lambda i: (0, 0)),
        ],
        out_specs=pl.BlockSpec((_BR, D), lambda i: (i, 0)),
        out_shape=jax.ShapeDtypeStruct((N_NODES, D), jnp.float32),
    )(x, W, b.reshape(1, D))


def _combine_mid(part, cnt1, xp, Wl, bl, Wr, g, be, Wp2, bp2):
    def body(p_ref, c_ref, xp_ref, wl_ref, bl_ref, wr_ref, g_ref, be_ref,
             wp_ref, bp_ref, o_ref):
        summed = p_ref[0] + p_ref[1]
        cnt = c_ref[0] + c_ref[1]
        mean = summed / jnp.maximum(cnt, 1.0)
        t = _dot(mean, wl_ref[...]) + bl_ref[...] + _dot(xp_ref[...], wr_ref[...])
        t = jnp.maximum(t, 0.0)
        mu = jnp.mean(t, axis=-1, keepdims=True)
        var = jnp.mean((t - mu) * (t - mu), axis=-1, keepdims=True)
        h = (t - mu) * lax.rsqrt(var + 1e-5) * g_ref[...] + be_ref[...]
        o_ref[...] = jnp.maximum(_dot(h, wp_ref[...]) + bp_ref[...], 0.0)

    return pl.pallas_call(
        body,
        grid=(N_NODES // _BR,),
        in_specs=[
            pl.BlockSpec((NC, _BR, D), lambda i: (0, i, 0)),
            pl.BlockSpec((NC, _BR, 1), lambda i: (0, i, 0)),
            pl.BlockSpec((_BR, D), lambda i: (i, 0)),
            pl.BlockSpec((D, D), lambda i: (0, 0)),
            pl.BlockSpec((1, D), lambda i: (0, 0)),
            pl.BlockSpec((D, D), lambda i: (0, 0)),
            pl.BlockSpec((1, D), lambda i: (0, 0)),
            pl.BlockSpec((1, D), lambda i: (0, 0)),
            pl.BlockSpec((D, D), lambda i: (0, 0)),
            pl.BlockSpec((1, D), lambda i: (0, 0)),
        ],
        out_specs=pl.BlockSpec((_BR, D), lambda i: (i, 0)),
        out_shape=jax.ShapeDtypeStruct((N_NODES, D), jnp.float32),
    )(part, cnt1, xp, Wl, bl.reshape(1, D), Wr, g.reshape(1, D),
      be.reshape(1, D), Wp2, bp2.reshape(1, D))


def _combine_final(part, cnt1, xp, Wl, bl, Wr):
    def body(p_ref, c_ref, xp_ref, wl_ref, bl_ref, wr_ref, o_ref):
        summed = p_ref[0] + p_ref[1]
        cnt = c_ref[0] + c_ref[1]
        mean = summed / jnp.maximum(cnt, 1.0)
        o_ref[...] = (_dot(mean, wl_ref[...]) + bl_ref[...]
                      + _dot(xp_ref[...], wr_ref[...]))

    return pl.pallas_call(
        body,
        grid=(N_NODES // _BR,),
        in_specs=[
            pl.BlockSpec((NC, _BR, D), lambda i: (0, i, 0)),
            pl.BlockSpec((NC, _BR, 1), lambda i: (0, i, 0)),
            pl.BlockSpec((_BR, D), lambda i: (i, 0)),
            pl.BlockSpec((D, D), lambda i: (0, 0)),
            pl.BlockSpec((1, D), lambda i: (0, 0)),
            pl.BlockSpec((D, D), lambda i: (0, 0)),
        ],
        out_specs=pl.BlockSpec((_BR, D), lambda i: (i, 0)),
        out_shape=jax.ShapeDtypeStruct((N_NODES, D), jnp.float32),
    )(part, cnt1, xp, Wl, bl.reshape(1, D), Wr)


def _unmap(a):
    return jnp.concatenate(
        [a[:, :GLO], a[:, GHI:GHI + (N_NODES - GLO)]], axis=1)


def kernel(x, edge_index, Wp1, bp1, Wl1, bl1, Wr1, g1, be1,
           Wp2, bp2, Wl2, bl2, Wr2):
    src = edge_index[0].astype(jnp.int32)
    dst = edge_index[1].astype(jnp.int32)
    dst2 = dst + jnp.where(dst >= GLO, 16, 0).astype(jnp.int32)
    zrow = jnp.zeros((CH, D), jnp.float32)
    ones = jnp.ones((CH, D), jnp.float32)

    cnt = _sc_count(dst2, zrow, ones)
    cnt1 = _unmap(cnt[:, :, :1])

    xp1 = _dense_relu(x, Wp1, bp1)
    part1 = _unmap(_sc_aggregate(xp1, src, dst2, zrow))
    xp2 = _combine_mid(part1, cnt1, xp1, Wl1, bl1, Wr1, g1, be1, Wp2, bp2)
    part2 = _unmap(_sc_aggregate(xp2, src, dst2, zrow))
    out = _combine_final(part2, cnt1, xp2, Wl2, bl2, Wr2)
    return out


# speedup vs baseline: 7.1552x; 1.1157x over previous
import functools

import jax
import jax.numpy as jnp
from jax import lax
from jax.experimental import pallas as pl
from jax.experimental.pallas import tpu as pltpu
from jax.experimental.pallas import tpu_sc as plsc

N_NODES = 10000
N_EDGES = 320000
D = 128

NC, NS = 2, 16
APAD = 10240
ZB = APAD // NS
CH = 80
HOLE = APAD // 2
GLO, GHI = HOLE - 8, HOLE + 8
EPC = N_EDGES // NC
EPT = EPC // NS
NCH = EPT // CH

_mesh = plsc.VectorSubcoreMesh(core_axis_name="c", subcore_axis_name="s")


@functools.partial(
    pl.kernel,
    out_type=jax.ShapeDtypeStruct((NC, APAD, D), jnp.float32),
    mesh=_mesh,
    scratch_types=[
        pltpu.VMEM((1, 2, CH), jnp.int32),
        pltpu.VMEM((1, 2, CH), jnp.int32),
        pltpu.VMEM((CH, D), jnp.float32),
        pltpu.VMEM((CH, D), jnp.float32),
        pltpu.VMEM_SHARED((APAD, D), jnp.float32),
        pltpu.SemaphoreType.DMA,
        pltpu.SemaphoreType.DMA,
    ],
)
def _sc_aggregate(xp_hbm, edges_hbm, zrow_hbm,
                  acc_out, ea_v, eb_v,
                  rows_a, rows_b, acc_sh, sem_a, sem_b):
    c = lax.axis_index("c")
    s = lax.axis_index("s")

    pltpu.sync_copy(zrow_hbm, rows_a)
    for k in range(ZB // CH):
        pltpu.sync_copy(rows_a, acc_sh.at[pl.ds(s * ZB + k * CH, CH)])
    plsc.subcore_barrier()

    cbase = (c * EPC + s * EPT) // CH

    pltpu.sync_copy(edges_hbm.at[pl.ds(cbase, 1)], ea_v)
    ga = pltpu.async_copy(xp_hbm.at[ea_v.at[0, 0]], rows_a, sem_a)

    @pl.loop(0, (NCH - 1) // 2)
    def _(j2):
        cid = cbase + 2 * j2
        pltpu.sync_copy(edges_hbm.at[pl.ds(cid + 1, 1)], eb_v)
        gb = pltpu.async_copy(xp_hbm.at[eb_v.at[0, 0]], rows_b, sem_b)
        ga.wait()
        pltpu.sync_copy(rows_a, acc_sh.at[ea_v.at[0, 1]], add=True)
        pltpu.sync_copy(edges_hbm.at[pl.ds(cid + 2, 1)], ea_v)
        pltpu.async_copy(xp_hbm.at[ea_v.at[0, 0]], rows_a, sem_a)
        gb.wait()
        pltpu.sync_copy(rows_b, acc_sh.at[eb_v.at[0, 1]], add=True)

    ga.wait()
    pltpu.sync_copy(rows_a, acc_sh.at[ea_v.at[0, 1]], add=True)

    plsc.subcore_barrier()
    for k in range(ZB // CH):
        pltpu.sync_copy(acc_sh.at[pl.ds(s * ZB + k * CH, CH)], rows_a)
        pltpu.sync_copy(rows_a, acc_out.at[c, pl.ds(s * ZB + k * CH, CH)])


@functools.partial(
    pl.kernel,
    out_type=jax.ShapeDtypeStruct((NC, APAD, D), jnp.float32),
    mesh=_mesh,
    scratch_types=[
        pltpu.VMEM((CH,), jnp.int32),
        pltpu.VMEM((CH, D), jnp.float32),
        pltpu.VMEM_SHARED((APAD, D), jnp.float32),
    ],
)
def _sc_count(dst_hbm, zrow_hbm, ones_hbm,
              cnt_out, dst_v, rows_v, cnt_sh):
    c = lax.axis_index("c")
    s = lax.axis_index("s")

    pltpu.sync_copy(zrow_hbm, rows_v)
    for k in range(ZB // CH):
        pltpu.sync_copy(rows_v, cnt_sh.at[pl.ds(s * ZB + k * CH, CH)])
    pltpu.sync_copy(ones_hbm, rows_v)
    plsc.subcore_barrier()

    ebase = c * EPC + s * EPT

    @pl.loop(0, NCH)
    def _(j):
        base = ebase + j * CH
        pltpu.sync_copy(dst_hbm.at[pl.ds(base, CH)], dst_v)
        pltpu.sync_copy(rows_v, cnt_sh.at[dst_v], add=True)

    plsc.subcore_barrier()
    for k in range(ZB // CH):
        pltpu.sync_copy(cnt_sh.at[pl.ds(s * ZB + k * CH, CH)], rows_v)
        pltpu.sync_copy(rows_v, cnt_out.at[c, pl.ds(s * ZB + k * CH, CH)])


_BR = 2000


def _dot(a, b):
    return jnp.dot(a, b, preferred_element_type=jnp.float32,
                   precision=lax.Precision.HIGHEST)


def _dense_relu(x, W, b):
    def body(x_ref, w_ref, b_ref, o_ref):
        o_ref[...] = jnp.maximum(_dot(x_ref[...], w_ref[...]) + b_ref[...], 0.0)

    return pl.pallas_call(
        body,
        grid=(N_NODES // _BR,),
        in_specs=[
            pl.BlockSpec((_BR, D), lambda i: (i, 0)),
            pl.BlockSpec((D, D), lambda i: (0, 0)),
            pl.BlockSpec((1, D), lambda i: (0, 0)),
        ],
        out_specs=pl.BlockSpec((_BR, D), lambda i: (i, 0)),
        out_shape=jax.ShapeDtypeStruct((N_NODES, D), jnp.float32),
    )(x, W, b.reshape(1, D))


def _combine_mid(part, cnt1, xp, Wl, bl, Wr, g, be, Wp2, bp2):
    def body(p_ref, c_ref, xp_ref, wl_ref, bl_ref, wr_ref, g_ref, be_ref,
             wp_ref, bp_ref, o_ref):
        summed = p_ref[0] + p_ref[1]
        cnt = c_ref[0] + c_ref[1]
        mean = summed / jnp.maximum(cnt, 1.0)
        t = _dot(mean, wl_ref[...]) + bl_ref[...] + _dot(xp_ref[...], wr_ref[...])
        t = jnp.maximum(t, 0.0)
        mu = jnp.mean(t, axis=-1, keepdims=True)
        var = jnp.mean((t - mu) * (t - mu), axis=-1, keepdims=True)
        h = (t - mu) * lax.rsqrt(var + 1e-5) * g_ref[...] + be_ref[...]
        o_ref[...] = jnp.maximum(_dot(h, wp_ref[...]) + bp_ref[...], 0.0)

    return pl.pallas_call(
        body,
        grid=(N_NODES // _BR,),
        in_specs=[
            pl.BlockSpec((NC, _BR, D), lambda i: (0, i, 0)),
            pl.BlockSpec((NC, _BR, 1), lambda i: (0, i, 0)),
            pl.BlockSpec((_BR, D), lambda i: (i, 0)),
            pl.BlockSpec((D, D), lambda i: (0, 0)),
            pl.BlockSpec((1, D), lambda i: (0, 0)),
            pl.BlockSpec((D, D), lambda i: (0, 0)),
            pl.BlockSpec((1, D), lambda i: (0, 0)),
            pl.BlockSpec((1, D), lambda i: (0, 0)),
            pl.BlockSpec((D, D), lambda i: (0, 0)),
            pl.BlockSpec((1, D), lambda i: (0, 0)),
        ],
        out_specs=pl.BlockSpec((_BR, D), lambda i: (i, 0)),
        out_shape=jax.ShapeDtypeStruct((N_NODES, D), jnp.float32),
    )(part, cnt1, xp, Wl, bl.reshape(1, D), Wr, g.reshape(1, D),
      be.reshape(1, D), Wp2, bp2.reshape(1, D))


def _combine_final(part, cnt1, xp, Wl, bl, Wr):
    def body(p_ref, c_ref, xp_ref, wl_ref, bl_ref, wr_ref, o_ref):
        summed = p_ref[0] + p_ref[1]
        cnt = c_ref[0] + c_ref[1]
        mean = summed / jnp.maximum(cnt, 1.0)
        o_ref[...] = (_dot(mean, wl_ref[...]) + bl_ref[...]
                      + _dot(xp_ref[...], wr_ref[...]))

    return pl.pallas_call(
        body,
        grid=(N_NODES // _BR,),
        in_specs=[
            pl.BlockSpec((NC, _BR, D), lambda i: (0, i, 0)),
            pl.BlockSpec((NC, _BR, 1), lambda i: (0, i, 0)),
            pl.BlockSpec((_BR, D), lambda i: (i, 0)),
            pl.BlockSpec((D, D), lambda i: (0, 0)),
            pl.BlockSpec((1, D), lambda i: (0, 0)),
            pl.BlockSpec((D, D), lambda i: (0, 0)),
        ],
        out_specs=pl.BlockSpec((_BR, D), lambda i: (i, 0)),
        out_shape=jax.ShapeDtypeStruct((N_NODES, D), jnp.float32),
    )(part, cnt1, xp, Wl, bl.reshape(1, D), Wr)


def _unmap(a):
    return jnp.concatenate(
        [a[:, :GLO], a[:, GHI:GHI + (N_NODES - GLO)]], axis=1)


def kernel(x, edge_index, Wp1, bp1, Wl1, bl1, Wr1, g1, be1,
           Wp2, bp2, Wl2, bl2, Wr2):
    src = edge_index[0].astype(jnp.int32)
    dst = edge_index[1].astype(jnp.int32)
    dst2 = dst + jnp.where(dst >= GLO, 16, 0).astype(jnp.int32)
    edges2d = jnp.stack(
        [src.reshape(N_EDGES // CH, CH), dst2.reshape(N_EDGES // CH, CH)],
        axis=1)
    zrow = jnp.zeros((CH, D), jnp.float32)
    ones = jnp.ones((CH, D), jnp.float32)

    cnt = _sc_count(dst2, zrow, ones)
    cnt1 = _unmap(cnt[:, :, :1])

    xp1 = _dense_relu(x, Wp1, bp1)
    part1 = _unmap(_sc_aggregate(xp1, edges2d, zrow))
    xp2 = _combine_mid(part1, cnt1, xp1, Wl1, bl1, Wr1, g1, be1, Wp2, bp2)
    part2 = _unmap(_sc_aggregate(xp2, edges2d, zrow))
    out = _combine_final(part2, cnt1, xp2, Wl2, bl2, Wr2)
    return out


# speedup vs baseline: 7.8328x; 1.0947x over previous
import functools

import jax
import jax.numpy as jnp
from jax import lax
from jax.experimental import pallas as pl
from jax.experimental.pallas import tpu as pltpu
from jax.experimental.pallas import tpu_sc as plsc

N_NODES = 10000
N_EDGES = 320000
D = 128

NC, NS = 2, 16
APAD = 10240
ZB = APAD // NS
CH = 80
HOLE = APAD // 2
GLO, GHI = HOLE - 8, HOLE + 8
EPC = N_EDGES // NC
EPT = EPC // NS
NCH = EPT // CH

_mesh = plsc.VectorSubcoreMesh(core_axis_name="c", subcore_axis_name="s")


@functools.partial(
    pl.kernel,
    out_type=jax.ShapeDtypeStruct((NC, APAD, D), jnp.float32),
    mesh=_mesh,
    scratch_types=[
        pltpu.VMEM((1, 2, CH), jnp.int32),
        pltpu.VMEM((1, 2, CH), jnp.int32),
        pltpu.VMEM((CH, D), jnp.float32),
        pltpu.VMEM((CH, D), jnp.float32),
        pltpu.VMEM_SHARED((APAD, D), jnp.float32),
        pltpu.SemaphoreType.DMA,
        pltpu.SemaphoreType.DMA,
    ],
)
def _sc_aggregate(xp_hbm, edges_hbm, zrow_hbm,
                  acc_out, ea_v, eb_v,
                  rows_a, rows_b, acc_sh, sem_a, sem_b):
    c = lax.axis_index("c")
    s = lax.axis_index("s")

    pltpu.sync_copy(zrow_hbm, rows_a)
    for k in range(ZB // CH):
        pltpu.sync_copy(rows_a, acc_sh.at[pl.ds(s * ZB + k * CH, CH)])
    plsc.subcore_barrier()

    cbase = (c * EPC + s * EPT) // CH

    pltpu.sync_copy(edges_hbm.at[pl.ds(cbase, 1)], ea_v)
    ga = pltpu.async_copy(xp_hbm.at[ea_v.at[0, 0]], rows_a, sem_a)

    @pl.loop(0, (NCH - 1) // 2)
    def _(j2):
        cid = cbase + 2 * j2
        pltpu.sync_copy(edges_hbm.at[pl.ds(cid + 1, 1)], eb_v)
        gb = pltpu.async_copy(xp_hbm.at[eb_v.at[0, 0]], rows_b, sem_b)
        ga.wait()
        pltpu.sync_copy(rows_a, acc_sh.at[ea_v.at[0, 1]], add=True)
        pltpu.sync_copy(edges_hbm.at[pl.ds(cid + 2, 1)], ea_v)
        pltpu.async_copy(xp_hbm.at[ea_v.at[0, 0]], rows_a, sem_a)
        gb.wait()
        pltpu.sync_copy(rows_b, acc_sh.at[eb_v.at[0, 1]], add=True)

    ga.wait()
    pltpu.sync_copy(rows_a, acc_sh.at[ea_v.at[0, 1]], add=True)

    plsc.subcore_barrier()
    for k in range(ZB // CH):
        pltpu.sync_copy(acc_sh.at[pl.ds(s * ZB + k * CH, CH)], rows_a)
        pltpu.sync_copy(rows_a, acc_out.at[c, pl.ds(s * ZB + k * CH, CH)])


@functools.partial(
    pl.kernel,
    out_type=jax.ShapeDtypeStruct((NC, APAD, D), jnp.float32),
    mesh=_mesh,
    scratch_types=[
        pltpu.VMEM((1, 2, CH), jnp.int32),
        pltpu.VMEM((1, 2, CH), jnp.int32),
        pltpu.VMEM((CH, D), jnp.float32),
        pltpu.VMEM_SHARED((APAD, D), jnp.float32),
        pltpu.SemaphoreType.DMA,
        pltpu.SemaphoreType.DMA,
    ],
)
def _sc_count(edges_hbm, zrow_hbm, ones_hbm,
              cnt_out, ea_v, eb_v, rows_v, cnt_sh, sem_a, sem_b):
    c = lax.axis_index("c")
    s = lax.axis_index("s")

    pltpu.sync_copy(zrow_hbm, rows_v)
    for k in range(ZB // CH):
        pltpu.sync_copy(rows_v, cnt_sh.at[pl.ds(s * ZB + k * CH, CH)])
    pltpu.sync_copy(ones_hbm, rows_v)
    plsc.subcore_barrier()

    cbase = (c * EPC + s * EPT) // CH

    ia = pltpu.async_copy(edges_hbm.at[pl.ds(cbase, 1)], ea_v, sem_a)

    @pl.loop(0, (NCH - 1) // 2)
    def _(j2):
        cid = cbase + 2 * j2
        ib = pltpu.async_copy(edges_hbm.at[pl.ds(cid + 1, 1)], eb_v, sem_b)
        ia.wait()
        pltpu.sync_copy(rows_v, cnt_sh.at[ea_v.at[0, 1]], add=True)
        pltpu.async_copy(edges_hbm.at[pl.ds(cid + 2, 1)], ea_v, sem_a)
        ib.wait()
        pltpu.sync_copy(rows_v, cnt_sh.at[eb_v.at[0, 1]], add=True)

    ia.wait()
    pltpu.sync_copy(rows_v, cnt_sh.at[ea_v.at[0, 1]], add=True)

    plsc.subcore_barrier()
    for k in range(ZB // CH):
        pltpu.sync_copy(cnt_sh.at[pl.ds(s * ZB + k * CH, CH)], rows_v)
        pltpu.sync_copy(rows_v, cnt_out.at[c, pl.ds(s * ZB + k * CH, CH)])


_BR = 2000


def _dot(a, b):
    return jnp.dot(a, b, preferred_element_type=jnp.float32,
                   precision=lax.Precision.HIGHEST)


def _dense_relu(x, W, b):
    def body(x_ref, w_ref, b_ref, o_ref):
        o_ref[...] = jnp.maximum(_dot(x_ref[...], w_ref[...]) + b_ref[...], 0.0)

    return pl.pallas_call(
        body,
        grid=(N_NODES // _BR,),
        in_specs=[
            pl.BlockSpec((_BR, D), lambda i: (i, 0)),
            pl.BlockSpec((D, D), lambda i: (0, 0)),
            pl.BlockSpec((1, D), lambda i: (0, 0)),
        ],
        out_specs=pl.BlockSpec((_BR, D), lambda i: (i, 0)),
        out_shape=jax.ShapeDtypeStruct((N_NODES, D), jnp.float32),
    )(x, W, b.reshape(1, D))


def _combine_mid(part, cnt1, xp, Wl, bl, Wr, g, be, Wp2, bp2):
    def body(p_ref, c_ref, xp_ref, wl_ref, bl_ref, wr_ref, g_ref, be_ref,
             wp_ref, bp_ref, o_ref):
        summed = p_ref[0] + p_ref[1]
        cnt = c_ref[0] + c_ref[1]
        mean = summed / jnp.maximum(cnt, 1.0)
        t = _dot(mean, wl_ref[...]) + bl_ref[...] + _dot(xp_ref[...], wr_ref[...])
        t = jnp.maximum(t, 0.0)
        mu = jnp.mean(t, axis=-1, keepdims=True)
        var = jnp.mean((t - mu) * (t - mu), axis=-1, keepdims=True)
        h = (t - mu) * lax.rsqrt(var + 1e-5) * g_ref[...] + be_ref[...]
        o_ref[...] = jnp.maximum(_dot(h, wp_ref[...]) + bp_ref[...], 0.0)

    return pl.pallas_call(
        body,
        grid=(N_NODES // _BR,),
        in_specs=[
            pl.BlockSpec((NC, _BR, D), lambda i: (0, i, 0)),
            pl.BlockSpec((NC, _BR, 1), lambda i: (0, i, 0)),
            pl.BlockSpec((_BR, D), lambda i: (i, 0)),
            pl.BlockSpec((D, D), lambda i: (0, 0)),
            pl.BlockSpec((1, D), lambda i: (0, 0)),
            pl.BlockSpec((D, D), lambda i: (0, 0)),
            pl.BlockSpec((1, D), lambda i: (0, 0)),
            pl.BlockSpec((1, D), lambda i: (0, 0)),
            pl.BlockSpec((D, D), lambda i: (0, 0)),
            pl.BlockSpec((1, D), lambda i: (0, 0)),
        ],
        out_specs=pl.BlockSpec((_BR, D), lambda i: (i, 0)),
        out_shape=jax.ShapeDtypeStruct((N_NODES, D), jnp.float32),
    )(part, cnt1, xp, Wl, bl.reshape(1, D), Wr, g.reshape(1, D),
      be.reshape(1, D), Wp2, bp2.reshape(1, D))


def _combine_final(part, cnt1, xp, Wl, bl, Wr):
    def body(p_ref, c_ref, xp_ref, wl_ref, bl_ref, wr_ref, o_ref):
        summed = p_ref[0] + p_ref[1]
        cnt = c_ref[0] + c_ref[1]
        mean = summed / jnp.maximum(cnt, 1.0)
        o_ref[...] = (_dot(mean, wl_ref[...]) + bl_ref[...]
                      + _dot(xp_ref[...], wr_ref[...]))

    return pl.pallas_call(
        body,
        grid=(N_NODES // _BR,),
        in_specs=[
            pl.BlockSpec((NC, _BR, D), lambda i: (0, i, 0)),
            pl.BlockSpec((NC, _BR, 1), lambda i: (0, i, 0)),
            pl.BlockSpec((_BR, D), lambda i: (i, 0)),
            pl.BlockSpec((D, D), lambda i: (0, 0)),
            pl.BlockSpec((1, D), lambda i: (0, 0)),
            pl.BlockSpec((D, D), lambda i: (0, 0)),
        ],
        out_specs=pl.BlockSpec((_BR, D), lambda i: (i, 0)),
        out_shape=jax.ShapeDtypeStruct((N_NODES, D), jnp.float32),
    )(part, cnt1, xp, Wl, bl.reshape(1, D), Wr)


def _unmap(a):
    return jnp.concatenate(
        [a[:, :GLO], a[:, GHI:GHI + (N_NODES - GLO)]], axis=1)


def kernel(x, edge_index, Wp1, bp1, Wl1, bl1, Wr1, g1, be1,
           Wp2, bp2, Wl2, bl2, Wr2):
    src = edge_index[0].astype(jnp.int32)
    dst = edge_index[1].astype(jnp.int32)
    dst2 = dst + jnp.where(dst >= GLO, 16, 0).astype(jnp.int32)
    edges2d = jnp.stack(
        [src.reshape(N_EDGES // CH, CH), dst2.reshape(N_EDGES // CH, CH)],
        axis=1)
    zrow = jnp.zeros((CH, D), jnp.float32)
    ones = jnp.ones((CH, D), jnp.float32)

    cnt = _sc_count(edges2d, zrow, ones)
    cnt1 = _unmap(cnt[:, :, :1])

    xp1 = _dense_relu(x, Wp1, bp1)
    part1 = _unmap(_sc_aggregate(xp1, edges2d, zrow))
    xp2 = _combine_mid(part1, cnt1, xp1, Wl1, bl1, Wr1, g1, be1, Wp2, bp2)
    part2 = _unmap(_sc_aggregate(xp2, edges2d, zrow))
    out = _combine_final(part2, cnt1, xp2, Wl2, bl2, Wr2)
    return out
